# baseline (speedup 1.0000x reference)
import jax
import jax.numpy as jnp
from jax import lax
from jax.experimental import pallas as pl
from jax.experimental.pallas import tpu as pltpu
from jax.experimental.pallas import tpu_sc as plsc

M = 500000
D = 128
HTS = 1 << 20
P0, P1, P2 = 73856093, 19349669, 83492791
NC, NS, L = 2, 16, 16
NW = NC * NS
C = 128
G_FULL = C // L
NCHUNK = M // C
K = NCHUNK // NW
NEXTRA = NCHUNK - K * NW
TAIL_ROWS = M - NCHUNK * C
TAIL_GROUPS = TAIL_ROWS // L
NPAIR = K // 2
RB = 4


def _grid_floor(q):
    d = q / jnp.float32(0.1)
    t = d.astype(jnp.int32)
    return jnp.where(t.astype(jnp.float32) > d, t - 1, t)


def _hash3(qx, qy, qz):
    gx, gy, gz = _grid_floor(qx), _grid_floor(qy), _grid_floor(qz)
    s = gx * P0 + gy * P1 + gz * P2
    return s & (HTS - 1)


def _sc_body(q_hbm, tab_hbm, feat_hbm, out_hbm,
             q0, hash_b, vox_b, sidx_b, rows0, rows1,
             hash_e, vox_e, sidx_e, hash_t, vox_t, sidx_t, tab_s,
             sem_q, sem_i, sem_g0, sem_g1, sem_w0, sem_w1, sem):
    wid = lax.axis_index("s") * NC + lax.axis_index("c")
    sid = lax.axis_index("s")
    lane = lax.iota(jnp.int32, L)
    c0 = jnp.zeros((L,), jnp.int32)
    c1 = c0 + 1
    c2 = c0 + 2

    def cbase(k):
        return (wid + k * NW) * C

    TSTRIPE = HTS // NS
    pltpu.sync_copy(tab_hbm.at[pl.ds(sid * TSTRIPE, TSTRIPE)],
                    tab_s.at[pl.ds(sid * TSTRIPE, TSTRIPE)])
    plsc.subcore_barrier()

    def slot(k):
        return (k & (RB - 1)) * C

    def q_start(k):
        pltpu.async_copy(q_hbm.at[pl.ds(cbase(k), C)], q0, sem_q)

    def q_wait():
        pltpu.make_async_copy(q_hbm.at[pl.ds(0, C)], q0, sem_q).wait()

    def hash_chunk(k):
        for g in range(G_FULL):
            rows16 = lane + g * L
            qx = plsc.load_gather(q0, [rows16, c0])
            qy = plsc.load_gather(q0, [rows16, c1])
            qz = plsc.load_gather(q0, [rows16, c2])
            hash_b[pl.ds(slot(k) + g * L, L)] = _hash3(qx, qy, qz)

    def idx_start(k):
        pltpu.async_copy(tab_s.at[hash_b.at[pl.ds(slot(k), C)]],
                         vox_b.at[pl.ds(slot(k), C)], sem_i)

    def idx_wait():
        pltpu.make_async_copy(tab_s.at[hash_b.at[pl.ds(0, C)]],
                              vox_b.at[pl.ds(0, C)], sem_i).wait()

    def clamp_chunk(k):
        for g in range(G_FULL):
            v = vox_b[pl.ds(slot(k) + g * L, L)]
            sidx_b[pl.ds(slot(k) + g * L, L)] = jnp.maximum(v, 0)

    def fire(k, rows_r, sem_g):
        pltpu.async_copy(feat_hbm.at[sidx_b.at[pl.ds(slot(k), C)]], rows_r,
                         sem_g)

    def fixup(k, rows_r):
        zeros = jnp.zeros((L,), jnp.float32)
        for g in range(G_FULL):
            vox = vox_b[pl.ds(slot(k) + g * L, L)]
            inv = vox < 0
            rows_idx = lane + g * L
            n_inv = jnp.max(inv.astype(jnp.int32), axis=0)

            @pl.when(n_inv > 0)
            def _():
                def zero_col(c, carry):
                    col = c0 + c
                    plsc.store_scatter(rows_r, [rows_idx, col], zeros,
                                       mask=inv)
                    return carry
                lax.fori_loop(jnp.int32(0), jnp.int32(D), zero_col,
                              jnp.int32(0))

    def drain(k, rows_r, sem_g, sem_w):
        pltpu.make_async_copy(feat_hbm.at[sidx_b.at[pl.ds(0, C)]], rows_r,
                              sem_g).wait()
        fixup(k, rows_r)
        pltpu.async_copy(rows_r, out_hbm.at[pl.ds(cbase(k), C)], sem_w)

    def w_wait(rows_r, sem_w):
        pltpu.make_async_copy(rows_r, out_hbm.at[pl.ds(0, C)], sem_w).wait()

    q_start(0)
    q_wait()
    hash_chunk(0)
    idx_start(0)
    q_start(1)
    q_wait()
    hash_chunk(1)
    idx_wait()
    idx_start(1)
    q_start(2)
    clamp_chunk(0)
    fire(0, rows0, sem_g0)

    def step(k, p, rows_r, sem_g, sem_w, rowsn_r, sem_gn, sem_wn, first):
        @pl.when(p < NPAIR - 1)
        def _():
            q_wait()
            hash_chunk(k + 2)
            idx_wait()
            idx_start(k + 2)

            @pl.when(k + 3 < K)
            def _():
                q_start(k + 3)
            clamp_chunk(k + 1)
            if not first:
                w_wait(rowsn_r, sem_wn)
            fire(k + 1, rowsn_r, sem_gn)

        @pl.when(p >= NPAIR - 1)
        def _():
            @pl.when(k + 1 < K)
            def _():
                idx_wait()
                clamp_chunk(k + 1)
                w_wait(rowsn_r, sem_wn)
                fire(k + 1, rowsn_r, sem_gn)
        drain(k, rows_r, sem_g, sem_w)

    def pair(p, carry):
        k = p * 2
        step(k, p, rows0, sem_g0, sem_w0, rows1, sem_g1, sem_w1, first=False)
        step(k + 1, p, rows1, sem_g1, sem_w1, rows0, sem_g0, sem_w0,
             first=False)
        return carry

    step(0, 0, rows0, sem_g0, sem_w0, rows1, sem_g1, sem_w1, first=True)
    step(1, 0, rows1, sem_g1, sem_w1, rows0, sem_g0, sem_w0, first=False)
    lax.fori_loop(jnp.int32(1), jnp.int32(NPAIR), pair, jnp.int32(0))
    w_wait(rows0, sem_w0)
    w_wait(rows1, sem_w1)

    def process(base, G, hash_r, vox_r, sidx_r):
        n = G * L
        pltpu.sync_copy(q_hbm.at[pl.ds(base, n)], q0.at[pl.ds(0, n)])
        for g in range(G):
            rows16 = lane + g * L
            qx = plsc.load_gather(q0, [rows16, c0])
            qy = plsc.load_gather(q0, [rows16, c1])
            qz = plsc.load_gather(q0, [rows16, c2])
            hash_r[pl.ds(g * L, L)] = _hash3(qx, qy, qz)
        pltpu.async_copy(tab_s.at[hash_r], vox_r, sem).wait()
        for g in range(G):
            vox = vox_r[pl.ds(g * L, L)]
            sidx_r[pl.ds(g * L, L)] = jnp.maximum(vox, 0)
        pltpu.async_copy(feat_hbm.at[sidx_r], rows0.at[pl.ds(0, n)],
                         sem).wait()
        zeros = jnp.zeros((L,), jnp.float32)
        for g in range(G):
            vox = vox_r[pl.ds(g * L, L)]
            inv = vox < 0
            rows_idx = lane + g * L
            n_inv = jnp.max(inv.astype(jnp.int32), axis=0)

            @pl.when(n_inv > 0)
            def _():
                def zero_col(c, carry):
                    col = c0 + c
                    plsc.store_scatter(rows0, [rows_idx, col], zeros,
                                       mask=inv)
                    return carry
                lax.fori_loop(jnp.int32(0), jnp.int32(D), zero_col,
                              jnp.int32(0))
        pltpu.sync_copy(rows0.at[pl.ds(0, n)], out_hbm.at[pl.ds(base, n)])

    @pl.when(wid < NEXTRA)
    def _():
        process((K * NW + wid) * C, G_FULL, hash_e, vox_e, sidx_e)

    @pl.when(wid < TAIL_GROUPS)
    def _():
        process(NCHUNK * C + wid * L, 1, hash_t, vox_t, sidx_t)


_mesh = plsc.VectorSubcoreMesh(core_axis_name="c", subcore_axis_name="s",
                               num_cores=NC, num_subcores=NS)

_sc_kernel = pl.kernel(
    _sc_body,
    out_type=jax.ShapeDtypeStruct((M, D), jnp.float32),
    mesh=_mesh,
    compiler_params=pltpu.CompilerParams(needs_layout_passes=False),
    scratch_types=[
        pltpu.VMEM((C, 3), jnp.float32),
        pltpu.VMEM((RB * C,), jnp.int32),
        pltpu.VMEM((RB * C,), jnp.int32),
        pltpu.VMEM((RB * C,), jnp.int32),
        pltpu.VMEM((C, D), jnp.float32),
        pltpu.VMEM((C, D), jnp.float32),
        pltpu.VMEM((C,), jnp.int32),
        pltpu.VMEM((C,), jnp.int32),
        pltpu.VMEM((C,), jnp.int32),
        pltpu.VMEM((L,), jnp.int32),
        pltpu.VMEM((L,), jnp.int32),
        pltpu.VMEM((L,), jnp.int32),
        pltpu.VMEM_SHARED((HTS,), jnp.int32),
        pltpu.SemaphoreType.DMA,
        pltpu.SemaphoreType.DMA,
        pltpu.SemaphoreType.DMA,
        pltpu.SemaphoreType.DMA,
        pltpu.SemaphoreType.DMA,
        pltpu.SemaphoreType.DMA,
        pltpu.SemaphoreType.DMA,
    ],
)


def kernel(query_pts, voxel_features, buffer_voxel_index):
    tab = buffer_voxel_index.astype(jnp.int32)
    return _sc_kernel(query_pts, tab, voxel_features)


# speedup vs baseline: 1.0059x; 1.0059x over previous
import jax
import jax.numpy as jnp
from jax import lax
from jax.experimental import pallas as pl
from jax.experimental.pallas import tpu as pltpu
from jax.experimental.pallas import tpu_sc as plsc

M = 500000
D = 128
HTS = 1 << 20
P0, P1, P2 = 73856093, 19349669, 83492791
NC, NS, L = 2, 16, 16
NW = NC * NS
C = 128
G_FULL = C // L
NCHUNK = M // C
K = NCHUNK // NW
NEXTRA = NCHUNK - K * NW
TAIL_ROWS = M - NCHUNK * C
TAIL_GROUPS = TAIL_ROWS // L
NPAIR = K // 2
RB = 4


def _grid_floor(q):
    d = q / jnp.float32(0.1)
    t = d.astype(jnp.int32)
    return jnp.where(t.astype(jnp.float32) > d, t - 1, t)


def _hash3(qx, qy, qz):
    gx, gy, gz = _grid_floor(qx), _grid_floor(qy), _grid_floor(qz)
    s = gx * P0 + gy * P1 + gz * P2
    return s & (HTS - 1)


def _sc_body(q_hbm, tab_hbm, feat_hbm, out_hbm,
             q0, hash_b, vox_b, sidx_b, rows0, rows1,
             hash_e, vox_e, sidx_e, hash_t, vox_t, sidx_t,
             sem_q, sem_i, sem_g0, sem_g1, sem_w0, sem_w1, sem):
    wid = lax.axis_index("s") * NC + lax.axis_index("c")
    lane = lax.iota(jnp.int32, L)
    c0 = jnp.zeros((L,), jnp.int32)
    c1 = c0 + 1
    c2 = c0 + 2

    def cbase(k):
        return (wid + k * NW) * C

    def slot(k):
        return (k & (RB - 1)) * C

    def q_start(k):
        pltpu.async_copy(q_hbm.at[pl.ds(cbase(k), C)], q0, sem_q)

    def q_wait():
        pltpu.make_async_copy(q_hbm.at[pl.ds(0, C)], q0, sem_q).wait()

    def hash_chunk(k):
        for g in range(G_FULL):
            rows16 = lane + g * L
            qx = plsc.load_gather(q0, [rows16, c0])
            qy = plsc.load_gather(q0, [rows16, c1])
            qz = plsc.load_gather(q0, [rows16, c2])
            hash_b[pl.ds(slot(k) + g * L, L)] = _hash3(qx, qy, qz)

    def idx_start(k):
        pltpu.async_copy(tab_hbm.at[hash_b.at[pl.ds(slot(k), C)]],
                         vox_b.at[pl.ds(slot(k), C)], sem_i)

    def idx_wait():
        pltpu.make_async_copy(tab_hbm.at[hash_b.at[pl.ds(0, C)]],
                              vox_b.at[pl.ds(0, C)], sem_i).wait()

    def clamp_chunk(k):
        for g in range(G_FULL):
            v = vox_b[pl.ds(slot(k) + g * L, L)]
            sidx_b[pl.ds(slot(k) + g * L, L)] = jnp.maximum(v, 0)

    def fire(k, rows_r, sem_g):
        pltpu.async_copy(feat_hbm.at[sidx_b.at[pl.ds(slot(k), C)]], rows_r,
                         sem_g)

    def fixup(k, rows_r):
        zeros = jnp.zeros((L,), jnp.float32)
        for g in range(G_FULL):
            vox = vox_b[pl.ds(slot(k) + g * L, L)]
            inv = vox < 0
            rows_idx = lane + g * L
            n_inv = jnp.max(inv.astype(jnp.int32), axis=0)

            @pl.when(n_inv > 0)
            def _():
                def zero_col(c, carry):
                    col = c0 + c
                    plsc.store_scatter(rows_r, [rows_idx, col], zeros,
                                       mask=inv)
                    return carry
                lax.fori_loop(jnp.int32(0), jnp.int32(D), zero_col,
                              jnp.int32(0))

    def drain(k, rows_r, sem_g, sem_w):
        pltpu.make_async_copy(feat_hbm.at[sidx_b.at[pl.ds(0, C)]], rows_r,
                              sem_g).wait()
        fixup(k, rows_r)
        pltpu.async_copy(rows_r, out_hbm.at[pl.ds(cbase(k), C)], sem_w)

    def w_wait(rows_r, sem_w):
        pltpu.make_async_copy(rows_r, out_hbm.at[pl.ds(0, C)], sem_w).wait()

    q_start(0)
    q_wait()
    hash_chunk(0)
    idx_start(0)
    q_start(1)
    q_wait()
    hash_chunk(1)
    idx_wait()
    idx_start(1)
    q_start(2)
    clamp_chunk(0)
    fire(0, rows0, sem_g0)

    def step(k, p, rows_r, sem_g, sem_w, rowsn_r, sem_gn, sem_wn, first):
        @pl.when(p < NPAIR - 1)
        def _():
            q_wait()
            hash_chunk(k + 2)
            idx_wait()
            idx_start(k + 2)

            @pl.when(k + 3 < K)
            def _():
                q_start(k + 3)
            clamp_chunk(k + 1)
            if not first:
                w_wait(rowsn_r, sem_wn)
            fire(k + 1, rowsn_r, sem_gn)

        @pl.when(p >= NPAIR - 1)
        def _():
            @pl.when(k + 1 < K)
            def _():
                idx_wait()
                clamp_chunk(k + 1)
                w_wait(rowsn_r, sem_wn)
                fire(k + 1, rowsn_r, sem_gn)
        drain(k, rows_r, sem_g, sem_w)

    def pair(p, carry):
        k = p * 2
        step(k, p, rows0, sem_g0, sem_w0, rows1, sem_g1, sem_w1, first=False)
        step(k + 1, p, rows1, sem_g1, sem_w1, rows0, sem_g0, sem_w0,
             first=False)
        return carry

    step(0, 0, rows0, sem_g0, sem_w0, rows1, sem_g1, sem_w1, first=True)
    step(1, 0, rows1, sem_g1, sem_w1, rows0, sem_g0, sem_w0, first=False)
    lax.fori_loop(jnp.int32(1), jnp.int32(NPAIR), pair, jnp.int32(0))
    w_wait(rows0, sem_w0)
    w_wait(rows1, sem_w1)

    def process(base, G, hash_r, vox_r, sidx_r):
        n = G * L
        pltpu.sync_copy(q_hbm.at[pl.ds(base, n)], q0.at[pl.ds(0, n)])
        for g in range(G):
            rows16 = lane + g * L
            qx = plsc.load_gather(q0, [rows16, c0])
            qy = plsc.load_gather(q0, [rows16, c1])
            qz = plsc.load_gather(q0, [rows16, c2])
            hash_r[pl.ds(g * L, L)] = _hash3(qx, qy, qz)
        pltpu.async_copy(tab_hbm.at[hash_r], vox_r, sem).wait()
        for g in range(G):
            vox = vox_r[pl.ds(g * L, L)]
            sidx_r[pl.ds(g * L, L)] = jnp.maximum(vox, 0)
        pltpu.async_copy(feat_hbm.at[sidx_r], rows0.at[pl.ds(0, n)],
                         sem).wait()
        zeros = jnp.zeros((L,), jnp.float32)
        for g in range(G):
            vox = vox_r[pl.ds(g * L, L)]
            inv = vox < 0
            rows_idx = lane + g * L
            n_inv = jnp.max(inv.astype(jnp.int32), axis=0)

            @pl.when(n_inv > 0)
            def _():
                def zero_col(c, carry):
                    col = c0 + c
                    plsc.store_scatter(rows0, [rows_idx, col], zeros,
                                       mask=inv)
                    return carry
                lax.fori_loop(jnp.int32(0), jnp.int32(D), zero_col,
                              jnp.int32(0))
        pltpu.sync_copy(rows0.at[pl.ds(0, n)], out_hbm.at[pl.ds(base, n)])

    @pl.when(wid < NEXTRA)
    def _():
        process((K * NW + wid) * C, G_FULL, hash_e, vox_e, sidx_e)

    @pl.when(wid < TAIL_GROUPS)
    def _():
        process(NCHUNK * C + wid * L, 1, hash_t, vox_t, sidx_t)


_mesh = plsc.VectorSubcoreMesh(core_axis_name="c", subcore_axis_name="s",
                               num_cores=NC, num_subcores=NS)

_sc_kernel = pl.kernel(
    _sc_body,
    out_type=jax.ShapeDtypeStruct((M, D), jnp.float32),
    mesh=_mesh,
    compiler_params=pltpu.CompilerParams(needs_layout_passes=False),
    scratch_types=[
        pltpu.VMEM((C, 3), jnp.float32),
        pltpu.VMEM((RB * C,), jnp.int32),
        pltpu.VMEM((RB * C,), jnp.int32),
        pltpu.VMEM((RB * C,), jnp.int32),
        pltpu.VMEM((C, D), jnp.float32),
        pltpu.VMEM((C, D), jnp.float32),
        pltpu.VMEM((C,), jnp.int32),
        pltpu.VMEM((C,), jnp.int32),
        pltpu.VMEM((C,), jnp.int32),
        pltpu.VMEM((L,), jnp.int32),
        pltpu.VMEM((L,), jnp.int32),
        pltpu.VMEM((L,), jnp.int32),
        pltpu.SemaphoreType.DMA,
        pltpu.SemaphoreType.DMA,
        pltpu.SemaphoreType.DMA,
        pltpu.SemaphoreType.DMA,
        pltpu.SemaphoreType.DMA,
        pltpu.SemaphoreType.DMA,
        pltpu.SemaphoreType.DMA,
    ],
)


def kernel(query_pts, voxel_features, buffer_voxel_index):
    tab = buffer_voxel_index.astype(jnp.int32)
    return _sc_kernel(query_pts, tab, voxel_features)
